# baseline (speedup 1.0000x reference)
import functools

import jax
import jax.numpy as jnp
from jax import lax
from jax.experimental import pallas as pl
from jax.experimental.pallas import tpu as pltpu
from jax.experimental.pallas import tpu_sc as plsc

_DX = 128
_DY = 26
_DYP = 32
_B = 4
_N = 512
_NEG = -1e30


def _emis_body(x_ref, w_ref, t_ref, emis_ref, tp_ref):
    e = jnp.dot(x_ref[0], w_ref[...], preferred_element_type=jnp.float32)
    pad = jnp.full((_N, _DYP - _DY), _NEG, jnp.float32)
    emis_ref[0] = jnp.concatenate([e, pad], axis=1)
    t_colpad = jnp.full((_DY, _DYP - _DY), _NEG, jnp.float32)
    t_rowpad = jnp.full((_DYP - _DY, _DYP), _NEG, jnp.float32)
    tp_ref[...] = jnp.concatenate(
        [jnp.concatenate([t_ref[...], t_colpad], axis=1), t_rowpad], axis=0)


def _compute_emis(X, W, T):
    return pl.pallas_call(
        _emis_body,
        grid=(_B,),
        in_specs=[
            pl.BlockSpec((1, _N, _DX), lambda b: (b, 0, 0)),
            pl.BlockSpec((_DX, _DY), lambda b: (0, 0)),
            pl.BlockSpec((_DY, _DY), lambda b: (0, 0)),
        ],
        out_specs=[
            pl.BlockSpec((1, _N, _DYP), lambda b: (b, 0, 0)),
            pl.BlockSpec((_DYP, _DYP), lambda b: (0, 0)),
        ],
        out_shape=[
            jax.ShapeDtypeStruct((_B, _N, _DYP), jnp.float32),
            jax.ShapeDtypeStruct((_DYP, _DYP), jnp.float32),
        ],
    )(X, W, T)


_sc_mesh = plsc.VectorSubcoreMesh(core_axis_name="c", subcore_axis_name="s")


@functools.partial(
    pl.kernel,
    mesh=_sc_mesh,
    out_type=jax.ShapeDtypeStruct((_B, _N * _DYP), jnp.float32),
    scratch_types=[
        pltpu.VMEM((_N * _DYP,), jnp.float32),
        pltpu.VMEM((_DYP * _DYP,), jnp.float32),
        pltpu.VMEM((_N * _DYP,), jnp.int32),
        pltpu.VMEM((_N * _DYP,), jnp.float32),
    ],
)
def _sc_decode(emis_hbm, t_hbm, out_hbm, emis_v, t_v, bp_v, out_v):
    c = lax.axis_index("c")
    s = lax.axis_index("s")
    w = c * 2 + s

    @pl.when(s < 2)
    def _():
        pltpu.sync_copy(emis_hbm.at[w], emis_v)
        pltpu.sync_copy(t_hbm, t_v)

        def fwd_step(i, carry):
            l0, l1 = carry
            e0 = emis_v[pl.ds((i - 1) * _DYP, 16)]
            e1 = emis_v[pl.ds((i - 1) * _DYP + 16, 16)]
            acc0 = jnp.full((16,), _NEG, jnp.float32)
            acc1 = jnp.full((16,), _NEG, jnp.float32)
            bp0 = jnp.zeros((16,), jnp.int32)
            bp1 = jnp.zeros((16,), jnp.int32)
            for y0 in range(_DY):
                xe = e0[y0] if y0 < 16 else e1[y0 - 16]
                xl = l0[y0] if y0 < 16 else l1[y0 - 16]
                t0 = t_v[pl.ds(y0 * _DYP, 16)]
                t1 = t_v[pl.ds(y0 * _DYP + 16, 16)]
                c0 = (xe + t0) + xl
                c1 = (xe + t1) + xl
                m0 = c0 > acc0
                m1 = c1 > acc1
                acc0 = jnp.where(m0, c0, acc0)
                acc1 = jnp.where(m1, c1, acc1)
                bp0 = jnp.where(m0, y0, bp0)
                bp1 = jnp.where(m1, y0, bp1)
            bp_v[pl.ds(i * _DYP, 16)] = bp0
            bp_v[pl.ds(i * _DYP + 16, 16)] = bp1
            return acc0, acc1

        zeros16 = jnp.zeros((16,), jnp.float32)
        l0, l1 = plsc.parallel_loop(
            1, _N, 1, unroll=2, carry=(zeros16, zeros16))(fwd_step)

        iota0 = lax.iota(jnp.int32, 16)
        iota1 = iota0 + 16

        def _butterfly(v, op):
            for sh in (8, 4, 2, 1):
                v = op(v, v.at[iota0 ^ sh].get(mode="promise_in_bounds"))
            return v

        v0 = emis_v[pl.ds((_N - 1) * _DYP, 16)] + l0
        v1 = emis_v[pl.ds((_N - 1) * _DYP + 16, 16)] + l1
        m = jnp.maximum(_butterfly(v0, jnp.maximum), _butterfly(v1, jnp.maximum))
        big = jnp.full((16,), _DYP, jnp.int32)
        a0 = jnp.where(v0 == m, iota0, big)
        a1 = jnp.where((v1 == m) & (iota1 < _DY), iota1, big)
        ans = _butterfly(jnp.minimum(a0, a1), jnp.minimum)

        one = jnp.float32(1.0)
        zero = jnp.float32(0.0)

        def write_row(i, a):
            out_v[pl.ds(i * _DYP, 16)] = jnp.where(iota0 == a, one, zero)
            out_v[pl.ds(i * _DYP + 16, 16)] = jnp.where(iota1 == a, one, zero)

        write_row(_N - 1, ans)

        fifteen = jnp.full((16,), 15, jnp.int32)

        def back_step(j, a):
            i = _N - 2 - j
            b0 = bp_v[pl.ds((i + 1) * _DYP, 16)]
            b1 = bp_v[pl.ds((i + 1) * _DYP + 16, 16)]
            g0 = b0.at[jnp.minimum(a, fifteen)].get(mode="promise_in_bounds")
            g1 = b1.at[jnp.maximum(a - 16, 0)].get(mode="promise_in_bounds")
            nxt = jnp.where(a < 16, g0, g1)
            write_row(i, nxt)
            return nxt

        lax.fori_loop(0, _N - 1, back_step, ans)

        pltpu.sync_copy(out_v, out_hbm.at[w])


def kernel(X, W, T):
    emis, tp = _compute_emis(X, W, T)
    out = _sc_decode(emis.reshape(_B, _N * _DYP), tp.reshape(_DYP * _DYP))
    return out.reshape(_B, _N, _DYP)[:, :, :_DY]


# speedup vs baseline: 1.0721x; 1.0721x over previous
import functools

import jax
import jax.numpy as jnp
from jax import lax
from jax.experimental import pallas as pl
from jax.experimental.pallas import tpu as pltpu
from jax.experimental.pallas import tpu_sc as plsc

_DX = 128
_DY = 26
_DYP = 32
_B = 4
_N = 512
_NEG = -1e30
_CH = 64
_NCH = _N // _CH
_HROWS = 10
_NPAD = 528
_WORD_W = _NPAD * _DYP


def _emis_body(x_ref, w_ref, t_ref, emis_ref, tp_ref):
    e = jnp.dot(x_ref[0], w_ref[...], preferred_element_type=jnp.float32)
    pad = jnp.full((_N, _DYP - _DY), _NEG, jnp.float32)
    emis_ref[0] = jnp.concatenate([e, pad], axis=1)
    t_colpad = jnp.full((_DY, _DYP - _DY), _NEG, jnp.float32)
    t_rowpad = jnp.full((_DYP - _DY, _DYP), _NEG, jnp.float32)
    tp_ref[...] = jnp.concatenate(
        [jnp.concatenate([t_ref[...], t_colpad], axis=1), t_rowpad], axis=0)


def _compute_emis(X, W, T):
    return pl.pallas_call(
        _emis_body,
        grid=(_B,),
        in_specs=[
            pl.BlockSpec((1, _N, _DX), lambda b: (b, 0, 0)),
            pl.BlockSpec((_DX, _DY), lambda b: (0, 0)),
            pl.BlockSpec((_DY, _DY), lambda b: (0, 0)),
        ],
        out_specs=[
            pl.BlockSpec((1, _N, _DYP), lambda b: (b, 0, 0)),
            pl.BlockSpec((_DYP, _DYP), lambda b: (0, 0)),
        ],
        out_shape=[
            jax.ShapeDtypeStruct((_B, _N, _DYP), jnp.float32),
            jax.ShapeDtypeStruct((_DYP, _DYP), jnp.float32),
        ],
    )(X, W, T)


_sc_mesh = plsc.VectorSubcoreMesh(core_axis_name="c", subcore_axis_name="s")


@functools.partial(
    pl.kernel,
    mesh=_sc_mesh,
    out_type=jax.ShapeDtypeStruct((_B, _N * _DYP), jnp.float32),
    scratch_types=[
        pltpu.VMEM((_NPAD * _DYP,), jnp.float32),
        pltpu.VMEM((_DYP * _DYP,), jnp.float32),
        pltpu.VMEM((_NPAD * _DYP,), jnp.float32),
        pltpu.VMEM((_N * _DYP,), jnp.int32),
        pltpu.VMEM((_N * _DYP,), jnp.float32),
        pltpu.VMEM(((_CH + 16) * _DYP,), jnp.float32),
        pltpu.VMEM_SHARED((2 * _WORD_W,), jnp.float32),
        pltpu.VMEM_SHARED((2 * _WORD_W,), jnp.int32),
    ],
)
def _sc_decode(emis_hbm, t_hbm, out_hbm, emis_v, t_v, lk_v, bp_v, out_v,
               lkc_v, lk_sh, bp_sh):
    c = lax.axis_index("c")
    s = lax.axis_index("s")
    wl = s // 8
    w = c * 2 + wl
    gs = s % 8
    leader = gs == 0
    sh_base = wl * _WORD_W

    pltpu.sync_copy(emis_hbm.at[w], emis_v.at[pl.ds(0, _N * _DYP)])
    pltpu.sync_copy(t_hbm, t_v)

    iota0 = lax.iota(jnp.int32, 16)
    iota1 = iota0 + 16
    zeros16 = jnp.zeros((16,), jnp.float32)

    @pl.when(leader)
    def _leader():
        lk_v[pl.ds(0, 16)] = zeros16
        lk_v[pl.ds(16, 16)] = zeros16

        def fwd_step(base):
            def step(j, carry):
                l0, l1 = carry
                i = base + j + 1
                e0 = emis_v[pl.ds((i - 1) * _DYP, 16)]
                e1 = emis_v[pl.ds((i - 1) * _DYP + 16, 16)]
                acc0 = jnp.full((16,), _NEG, jnp.float32)
                acc1 = jnp.full((16,), _NEG, jnp.float32)
                for y0 in range(_DY):
                    xe = e0[y0] if y0 < 16 else e1[y0 - 16]
                    xl = l0[y0] if y0 < 16 else l1[y0 - 16]
                    t0 = t_v[pl.ds(y0 * _DYP, 16)]
                    t1 = t_v[pl.ds(y0 * _DYP + 16, 16)]
                    acc0 = jnp.maximum(acc0, (xe + t0) + xl)
                    acc1 = jnp.maximum(acc1, (xe + t1) + xl)
                lk_v[pl.ds(i * _DYP, 16)] = acc0
                lk_v[pl.ds(i * _DYP + 16, 16)] = acc1
                return acc0, acc1
            return step

        def chunk(k, carry):
            base = k * _CH
            nsteps = jnp.minimum(_CH, (_N - 1) - base)
            carry = lax.fori_loop(0, nsteps, fwd_step(base), carry)
            pltpu.sync_copy(
                lk_v.at[pl.ds(base * _DYP, (_CH + 1) * _DYP)],
                lk_sh.at[pl.ds(sh_base + base * _DYP, (_CH + 1) * _DYP)])
            plsc.subcore_barrier()
            return carry

        l0, l1 = lax.fori_loop(0, _NCH, chunk, (zeros16, zeros16))
        plsc.subcore_barrier()

        pltpu.sync_copy(bp_sh.at[pl.ds(sh_base, _N * _DYP)], bp_v)

        def _butterfly(v, op):
            for sh in (8, 4, 2, 1):
                v = op(v, v.at[iota0 ^ sh].get(mode="promise_in_bounds"))
            return v

        v0 = emis_v[pl.ds((_N - 1) * _DYP, 16)] + l0
        v1 = emis_v[pl.ds((_N - 1) * _DYP + 16, 16)] + l1
        m = jnp.maximum(_butterfly(v0, jnp.maximum),
                        _butterfly(v1, jnp.maximum))
        big = jnp.full((16,), _DYP, jnp.int32)
        a0 = jnp.where(v0 == m, iota0, big)
        a1 = jnp.where((v1 == m) & (iota1 < _DY), iota1, big)
        ans = _butterfly(jnp.minimum(a0, a1), jnp.minimum)

        one = jnp.float32(1.0)
        zero = jnp.float32(0.0)

        def write_row(i, a):
            out_v[pl.ds(i * _DYP, 16)] = jnp.where(iota0 == a, one, zero)
            out_v[pl.ds(i * _DYP + 16, 16)] = jnp.where(iota1 == a, one, zero)

        write_row(_N - 1, ans)

        fifteen = jnp.full((16,), 15, jnp.int32)

        def back_step(j, a):
            i = _N - 2 - j
            b0 = bp_v[pl.ds((i + 1) * _DYP, 16)]
            b1 = bp_v[pl.ds((i + 1) * _DYP + 16, 16)]
            g0 = b0.at[jnp.minimum(a, fifteen)].get(mode="promise_in_bounds")
            g1 = b1.at[jnp.maximum(a - 16, 0)].get(mode="promise_in_bounds")
            nxt = jnp.where(a < 16, g0, g1)
            write_row(i, nxt)
            return nxt

        lax.fori_loop(0, _N - 1, back_step, ans)

        pltpu.sync_copy(out_v, out_hbm.at[w])

    @pl.when(jnp.logical_not(leader))
    def _helper():
        h = gs - 1

        def bp_row(base):
            def row(r, carry):
                i = base + 1 + h * _HROWS + r
                loc = i - base
                e0 = emis_v[pl.ds((i - 1) * _DYP, 16)]
                e1 = emis_v[pl.ds((i - 1) * _DYP + 16, 16)]
                ln0 = lkc_v[pl.ds(loc * _DYP, 16)]
                ln1 = lkc_v[pl.ds(loc * _DYP + 16, 16)]
                l0 = lkc_v[pl.ds((loc - 1) * _DYP, 16)]
                l1 = lkc_v[pl.ds((loc - 1) * _DYP + 16, 16)]
                bp0 = jnp.full((16,), _DYP, jnp.int32)
                bp1 = jnp.full((16,), _DYP, jnp.int32)
                for y0 in range(_DY):
                    xe = e0[y0] if y0 < 16 else e1[y0 - 16]
                    xl = l0[y0] if y0 < 16 else l1[y0 - 16]
                    t0 = t_v[pl.ds(y0 * _DYP, 16)]
                    t1 = t_v[pl.ds(y0 * _DYP + 16, 16)]
                    c0 = (xe + t0) + xl
                    c1 = (xe + t1) + xl
                    bp0 = jnp.minimum(bp0, jnp.where(c0 == ln0, y0, _DYP))
                    bp1 = jnp.minimum(bp1, jnp.where(c1 == ln1, y0, _DYP))
                bp_v[pl.ds(r * _DYP, 16)] = bp0
                bp_v[pl.ds(r * _DYP + 16, 16)] = bp1
                return carry
            return row

        def chunk(k, carry):
            base = k * _CH
            plsc.subcore_barrier()
            pltpu.sync_copy(
                lk_sh.at[pl.ds(sh_base + base * _DYP, (_CH + 1) * _DYP)],
                lkc_v.at[pl.ds(0, (_CH + 1) * _DYP)])
            lax.fori_loop(0, _HROWS, bp_row(base), 0)
            pltpu.sync_copy(
                bp_v.at[pl.ds(0, _HROWS * _DYP)],
                bp_sh.at[pl.ds(
                    sh_base + (base + 1 + h * _HROWS) * _DYP,
                    _HROWS * _DYP)])
            return carry

        lax.fori_loop(0, _NCH, chunk, 0)
        plsc.subcore_barrier()


def kernel(X, W, T):
    emis, tp = _compute_emis(X, W, T)
    out = _sc_decode(emis.reshape(_B, _N * _DYP), tp.reshape(_DYP * _DYP))
    return out.reshape(_B, _N, _DYP)[:, :, :_DY]
